# baseline (speedup 1.0000x reference)
import functools

import jax
import jax.numpy as jnp
from jax import lax
from jax.experimental import pallas as pl
from jax.experimental.pallas import tpu as pltpu
from jax.experimental.pallas import tpu_sc as plsc

SMOOTH = 0.1
L = 16


def _token_stats(buf, label_vec, num_classes, unroll):
    nv = num_classes // L
    n_it = nv // unroll

    def body_a(i, carry):
        ms = list(carry)
        for u in range(unroll):
            x = buf[pl.ds((i * unroll + u) * L, L)]
            ms[u] = jnp.maximum(ms[u], x)
        return tuple(ms)

    m_init = tuple(jnp.full((L,), -jnp.inf, jnp.float32) for _ in range(unroll))
    ms = lax.fori_loop(0, n_it, body_a, m_init)
    mv = functools.reduce(jnp.maximum, ms)
    m = jnp.max(mv)
    mb = jnp.full((L,), m, jnp.float32)

    def body_b(i, carry):
        ss = list(carry[:unroll])
        es = list(carry[unroll:])
        for u in range(unroll):
            x = buf[pl.ds((i * unroll + u) * L, L)]
            ss[u] = ss[u] + x
            es[u] = es[u] + jnp.exp(x - mb)
        return tuple(ss) + tuple(es)

    z = tuple(jnp.zeros((L,), jnp.float32) for _ in range(2 * unroll))
    r = lax.fori_loop(0, n_it, body_b, z)
    S = jnp.sum(functools.reduce(jnp.add, r[:unroll]))
    E = jnp.sum(functools.reduce(jnp.add, r[unroll:]))
    xl = jnp.max(plsc.load_gather(buf, [label_vec]))
    return m, S, E, xl


def _store1(res_v, idx, val, lane0):
    plsc.store_scatter(res_v, [jnp.full((L,), idx, jnp.int32)],
                       jnp.full((L,), val, jnp.float32), mask=lane0)


def _sc_body(n_tokens, num_classes, tpw, unroll,
             preds_hbm, labels_hbm, m_hbm, s_hbm, e_hbm, x_hbm,
             row0, row1, lab_v, res_v, sem0, sem1):
    c = lax.axis_index("c")
    s = lax.axis_index("s")
    info = plsc.get_sparse_core_info()
    wid = s * info.num_cores + c
    base = wid * tpw
    lane = jnp.arange(L, dtype=jnp.int32)
    lane0 = lane == 0

    pltpu.sync_copy(labels_hbm.at[pl.ds(base, tpw)], lab_v)

    def clamped(t):
        return jnp.minimum(t, n_tokens - 1)

    pltpu.async_copy(preds_hbm.at[base], row0, sem0)
    pltpu.async_copy(preds_hbm.at[base + 1], row1, sem1)

    def handle_token(k, buf, sem, prefetch_t):
        pltpu.make_async_copy(preds_hbm.at[clamped(base + k)], buf, sem).wait()
        label_vec = plsc.load_gather(lab_v, [jnp.full((L,), k, jnp.int32)])
        m, S, E, xl = _token_stats(buf, label_vec, num_classes, unroll)
        pltpu.async_copy(preds_hbm.at[clamped(prefetch_t)], buf, sem)
        _store1(res_v, k, m, lane0)
        _store1(res_v, tpw + k, S, lane0)
        _store1(res_v, 2 * tpw + k, E, lane0)
        _store1(res_v, 3 * tpw + k, xl, lane0)

    def body(j, carry):
        k0 = 2 * j
        handle_token(k0, row0, sem0, base + k0 + 2)
        handle_token(k0 + 1, row1, sem1, base + k0 + 3)
        return carry

    lax.fori_loop(0, tpw // 2, body, jnp.int32(0))
    pltpu.make_async_copy(preds_hbm.at[0], row0, sem0).wait()
    pltpu.make_async_copy(preds_hbm.at[0], row1, sem1).wait()

    for q, out in enumerate((m_hbm, s_hbm, e_hbm, x_hbm)):
        pltpu.sync_copy(res_v.at[pl.ds(q * tpw, tpw)], out.at[pl.ds(base, tpw)])


def _combine_block(m_ref, s_ref, e_ref, x_ref, out_ref, *, n_tokens,
                   num_classes):
    m = m_ref[...]
    total = s_ref[...]
    sumexp = e_ref[...]
    xl = x_ref[...]
    lse = m + jnp.log(sumexp)
    a = SMOOTH / (num_classes - 1)
    lp_label = xl - lse
    sum_lp = total - num_classes * lse
    loss_t = -(a * (sum_lp - lp_label) + (1.0 - SMOOTH) * lp_label)
    out_ref[...] = jnp.sum(loss_t, keepdims=True).reshape(1, 1) / n_tokens


def kernel(preds, labels):
    b, t, c = preds.shape
    n_tokens = b * t
    preds2 = preds.reshape(n_tokens, c)
    labels1 = labels.reshape(n_tokens).astype(jnp.int32)

    info = plsc.get_sparse_core_info()
    n_workers = info.num_cores * info.num_subcores
    tpw = n_tokens // n_workers
    unroll = 8

    mesh = plsc.VectorSubcoreMesh(core_axis_name="c", subcore_axis_name="s")
    out_t = jax.ShapeDtypeStruct((n_tokens,), jnp.float32)
    sc = pl.kernel(
        functools.partial(_sc_body, n_tokens, c, tpw, unroll),
        out_type=(out_t, out_t, out_t, out_t),
        mesh=mesh,
        compiler_params=pltpu.CompilerParams(needs_layout_passes=False),
        scratch_types=[
            pltpu.VMEM((c,), jnp.float32),
            pltpu.VMEM((c,), jnp.float32),
            pltpu.VMEM((tpw,), jnp.int32),
            pltpu.VMEM((4 * tpw,), jnp.float32),
            pltpu.SemaphoreType.DMA,
            pltpu.SemaphoreType.DMA,
        ],
    )
    m_a, s_a, e_a, x_a = sc(preds2, labels1)

    shp = (n_tokens // 128, 128)
    out = pl.pallas_call(
        functools.partial(_combine_block, n_tokens=n_tokens, num_classes=c),
        out_shape=jax.ShapeDtypeStruct((1, 1), jnp.float32),
    )(m_a.reshape(shp), s_a.reshape(shp), e_a.reshape(shp), x_a.reshape(shp))
    return out[0, 0]


# speedup vs baseline: 1.0163x; 1.0163x over previous
import functools

import jax
import jax.numpy as jnp
from jax import lax
from jax.experimental import pallas as pl
from jax.experimental.pallas import tpu as pltpu
from jax.experimental.pallas import tpu_sc as plsc

SMOOTH = 0.1
L = 16


def _token_stats(buf, label_vec, num_classes, unroll):
    nv = num_classes // L
    n_it = nv // unroll

    def body_a(i, carry):
        ms = list(carry)
        for u in range(unroll):
            x = buf[pl.ds((i * unroll + u) * L, L)]
            ms[u] = jnp.maximum(ms[u], x)
        return tuple(ms)

    m_init = tuple(jnp.full((L,), -jnp.inf, jnp.float32) for _ in range(unroll))
    ms = lax.fori_loop(0, n_it, body_a, m_init)
    mv = functools.reduce(jnp.maximum, ms)
    m = jnp.max(mv)
    mb = jnp.full((L,), m, jnp.float32)

    def body_b(i, carry):
        ss = list(carry[:unroll])
        es = list(carry[unroll:])
        for u in range(unroll):
            x = buf[pl.ds((i * unroll + u) * L, L)]
            ss[u] = ss[u] + x
            es[u] = es[u] + jnp.exp(x - mb)
        return tuple(ss) + tuple(es)

    z = tuple(jnp.zeros((L,), jnp.float32) for _ in range(2 * unroll))
    r = lax.fori_loop(0, n_it, body_b, z)
    S = jnp.sum(functools.reduce(jnp.add, r[:unroll]))
    E = jnp.sum(functools.reduce(jnp.add, r[unroll:]))
    xl = jnp.max(plsc.load_gather(buf, [label_vec]))
    return m, S, E, xl


def _store1(res_v, idx, val, lane0):
    plsc.store_scatter(res_v, [jnp.full((L,), idx, jnp.int32)],
                       jnp.full((L,), val, jnp.float32), mask=lane0)


def _sc_body(n_tokens, num_classes, tpw, unroll,
             preds_hbm, labels_hbm, m_hbm, s_hbm, e_hbm, x_hbm,
             row0, row1, lab_v, res_v, sem0, sem1):
    c = lax.axis_index("c")
    s = lax.axis_index("s")
    info = plsc.get_sparse_core_info()
    wid = s * info.num_cores + c
    base = wid * tpw
    lane = jnp.arange(L, dtype=jnp.int32)
    lane0 = lane == 0

    pltpu.sync_copy(labels_hbm.at[pl.ds(base, tpw)], lab_v)

    def clamped(t):
        return jnp.minimum(t, n_tokens - 1)

    pltpu.async_copy(preds_hbm.at[base], row0, sem0)
    pltpu.async_copy(preds_hbm.at[base + 1], row1, sem1)

    def handle_token(k, buf, sem, prefetch_t):
        pltpu.make_async_copy(preds_hbm.at[clamped(base + k)], buf, sem).wait()
        label_vec = plsc.load_gather(lab_v, [jnp.full((L,), k, jnp.int32)])
        m, S, E, xl = _token_stats(buf, label_vec, num_classes, unroll)
        pltpu.async_copy(preds_hbm.at[clamped(prefetch_t)], buf, sem)
        _store1(res_v, k, m, lane0)
        _store1(res_v, tpw + k, S, lane0)
        _store1(res_v, 2 * tpw + k, E, lane0)
        _store1(res_v, 3 * tpw + k, xl, lane0)

    def body(j, carry):
        k0 = 2 * j
        handle_token(k0, row0, sem0, base + k0 + 2)
        handle_token(k0 + 1, row1, sem1, base + k0 + 3)
        return carry

    lax.fori_loop(0, tpw // 2, body, jnp.int32(0))
    pltpu.make_async_copy(preds_hbm.at[0], row0, sem0).wait()
    pltpu.make_async_copy(preds_hbm.at[0], row1, sem1).wait()

    for q, out in enumerate((m_hbm, s_hbm, e_hbm, x_hbm)):
        pltpu.sync_copy(res_v.at[pl.ds(q * tpw, tpw)], out.at[pl.ds(base, tpw)])


def _combine_block(m_ref, s_ref, e_ref, x_ref, out_ref, *, n_tokens,
                   num_classes):
    m = m_ref[...]
    total = s_ref[...]
    sumexp = e_ref[...]
    xl = x_ref[...]
    lse = m + jnp.log(sumexp)
    a = SMOOTH / (num_classes - 1)
    lp_label = xl - lse
    sum_lp = total - num_classes * lse
    loss_t = -(a * (sum_lp - lp_label) + (1.0 - SMOOTH) * lp_label)
    out_ref[...] = jnp.sum(loss_t, keepdims=True).reshape(1, 1) / n_tokens


def kernel(preds, labels):
    b, t, c = preds.shape
    n_tokens = b * t
    preds2 = preds.reshape(n_tokens, c)
    labels1 = labels.reshape(n_tokens).astype(jnp.int32)

    info = plsc.get_sparse_core_info()
    n_workers = info.num_cores * info.num_subcores
    tpw = n_tokens // n_workers
    unroll = 16

    mesh = plsc.VectorSubcoreMesh(core_axis_name="c", subcore_axis_name="s")
    out_t = jax.ShapeDtypeStruct((n_tokens,), jnp.float32)
    sc = pl.kernel(
        functools.partial(_sc_body, n_tokens, c, tpw, unroll),
        out_type=(out_t, out_t, out_t, out_t),
        mesh=mesh,
        compiler_params=pltpu.CompilerParams(needs_layout_passes=False),
        scratch_types=[
            pltpu.VMEM((c,), jnp.float32),
            pltpu.VMEM((c,), jnp.float32),
            pltpu.VMEM((tpw,), jnp.int32),
            pltpu.VMEM((4 * tpw,), jnp.float32),
            pltpu.SemaphoreType.DMA,
            pltpu.SemaphoreType.DMA,
        ],
    )
    m_a, s_a, e_a, x_a = sc(preds2, labels1)

    shp = (n_tokens // 128, 128)
    out = pl.pallas_call(
        functools.partial(_combine_block, n_tokens=n_tokens, num_classes=c),
        out_shape=jax.ShapeDtypeStruct((1, 1), jnp.float32),
    )(m_a.reshape(shp), s_a.reshape(shp), e_a.reshape(shp), x_a.reshape(shp))
    return out[0, 0]


# speedup vs baseline: 1.0431x; 1.0263x over previous
import functools

import jax
import jax.numpy as jnp
from jax import lax
from jax.experimental import pallas as pl
from jax.experimental.pallas import tpu as pltpu
from jax.experimental.pallas import tpu_sc as plsc

SMOOTH = 0.1
L = 16


def _token_stats(buf, label_vec, num_classes, unroll):
    nv = num_classes // L
    n_it = nv // unroll

    m_init = tuple(jnp.full((L,), -jnp.inf, jnp.float32) for _ in range(unroll))
    s_init = tuple(jnp.zeros((L,), jnp.float32) for _ in range(unroll))

    @plsc.parallel_loop(0, n_it, carry=m_init + s_init)
    def pass_a(i, carry):
        ms = list(carry[:unroll])
        ss = list(carry[unroll:])
        for u in range(unroll):
            x = buf[pl.ds((i * unroll + u) * L, L)]
            ms[u] = jnp.maximum(ms[u], x)
            ss[u] = ss[u] + x
        return tuple(ms) + tuple(ss)

    mv = functools.reduce(jnp.maximum, pass_a[:unroll])
    S = jnp.sum(functools.reduce(jnp.add, pass_a[unroll:]))
    m = jnp.max(mv)
    mb = jnp.full((L,), m, jnp.float32)

    @plsc.parallel_loop(0, n_it, carry=s_init)
    def pass_b(i, carry):
        es = list(carry)
        for u in range(unroll):
            x = buf[pl.ds((i * unroll + u) * L, L)]
            es[u] = es[u] + jnp.exp(x - mb)
        return tuple(es)

    E = jnp.sum(functools.reduce(jnp.add, pass_b))
    xl = jnp.max(plsc.load_gather(buf, [label_vec]))
    return m, S, E, xl


def _store1(res_v, idx, val, lane0):
    plsc.store_scatter(res_v, [jnp.full((L,), idx, jnp.int32)],
                       jnp.full((L,), val, jnp.float32), mask=lane0)


def _sc_body(n_tokens, num_classes, tpw, unroll,
             preds_hbm, labels_hbm, m_hbm, s_hbm, e_hbm, x_hbm,
             row0, row1, lab_v, res_v, sem0, sem1):
    c = lax.axis_index("c")
    s = lax.axis_index("s")
    info = plsc.get_sparse_core_info()
    wid = s * info.num_cores + c
    base = wid * tpw
    lane = jnp.arange(L, dtype=jnp.int32)
    lane0 = lane == 0

    pltpu.sync_copy(labels_hbm.at[pl.ds(base, tpw)], lab_v)

    def clamped(t):
        return jnp.minimum(t, n_tokens - 1)

    pltpu.async_copy(preds_hbm.at[base], row0, sem0)
    pltpu.async_copy(preds_hbm.at[base + 1], row1, sem1)

    def handle_token(k, buf, sem, prefetch_t):
        pltpu.make_async_copy(preds_hbm.at[clamped(base + k)], buf, sem).wait()
        label_vec = plsc.load_gather(lab_v, [jnp.full((L,), k, jnp.int32)])
        m, S, E, xl = _token_stats(buf, label_vec, num_classes, unroll)
        pltpu.async_copy(preds_hbm.at[clamped(prefetch_t)], buf, sem)
        _store1(res_v, k, m, lane0)
        _store1(res_v, tpw + k, S, lane0)
        _store1(res_v, 2 * tpw + k, E, lane0)
        _store1(res_v, 3 * tpw + k, xl, lane0)

    def body(j, carry):
        k0 = 2 * j
        handle_token(k0, row0, sem0, base + k0 + 2)
        handle_token(k0 + 1, row1, sem1, base + k0 + 3)
        return carry

    lax.fori_loop(0, tpw // 2, body, jnp.int32(0))
    pltpu.make_async_copy(preds_hbm.at[0], row0, sem0).wait()
    pltpu.make_async_copy(preds_hbm.at[0], row1, sem1).wait()

    for q, out in enumerate((m_hbm, s_hbm, e_hbm, x_hbm)):
        pltpu.sync_copy(res_v.at[pl.ds(q * tpw, tpw)], out.at[pl.ds(base, tpw)])


def _combine_block(m_ref, s_ref, e_ref, x_ref, out_ref, *, n_tokens,
                   num_classes):
    m = m_ref[...]
    total = s_ref[...]
    sumexp = e_ref[...]
    xl = x_ref[...]
    lse = m + jnp.log(sumexp)
    a = SMOOTH / (num_classes - 1)
    lp_label = xl - lse
    sum_lp = total - num_classes * lse
    loss_t = -(a * (sum_lp - lp_label) + (1.0 - SMOOTH) * lp_label)
    out_ref[...] = jnp.sum(loss_t, keepdims=True).reshape(1, 1) / n_tokens


def kernel(preds, labels):
    b, t, c = preds.shape
    n_tokens = b * t
    preds2 = preds.reshape(n_tokens, c)
    labels1 = labels.reshape(n_tokens).astype(jnp.int32)

    info = plsc.get_sparse_core_info()
    n_workers = info.num_cores * info.num_subcores
    tpw = n_tokens // n_workers
    unroll = 8

    mesh = plsc.VectorSubcoreMesh(core_axis_name="c", subcore_axis_name="s")
    out_t = jax.ShapeDtypeStruct((n_tokens,), jnp.float32)
    sc = pl.kernel(
        functools.partial(_sc_body, n_tokens, c, tpw, unroll),
        out_type=(out_t, out_t, out_t, out_t),
        mesh=mesh,
        compiler_params=pltpu.CompilerParams(needs_layout_passes=False),
        scratch_types=[
            pltpu.VMEM((c,), jnp.float32),
            pltpu.VMEM((c,), jnp.float32),
            pltpu.VMEM((tpw,), jnp.int32),
            pltpu.VMEM((4 * tpw,), jnp.float32),
            pltpu.SemaphoreType.DMA,
            pltpu.SemaphoreType.DMA,
        ],
    )
    m_a, s_a, e_a, x_a = sc(preds2, labels1)

    shp = (n_tokens // 128, 128)
    out = pl.pallas_call(
        functools.partial(_combine_block, n_tokens=n_tokens, num_classes=c),
        out_shape=jax.ShapeDtypeStruct((1, 1), jnp.float32),
    )(m_a.reshape(shp), s_a.reshape(shp), e_a.reshape(shp), x_a.reshape(shp))
    return out[0, 0]


# speedup vs baseline: 2.1211x; 2.0335x over previous
import functools

import jax
import jax.numpy as jnp
from jax import lax
from jax.experimental import pallas as pl
from jax.experimental.pallas import tpu as pltpu
from jax.experimental.pallas import tpu_sc as plsc

SMOOTH = 0.1
L = 16


def _token_stats(buf, label_vec, n_cols, unroll):
    n_it = n_cols // L // unroll

    m_init = tuple(jnp.full((L,), -jnp.inf, jnp.float32) for _ in range(unroll))
    s_init = tuple(jnp.zeros((L,), jnp.float32) for _ in range(unroll))

    @plsc.parallel_loop(0, n_it, carry=m_init + s_init)
    def pass_a(i, carry):
        ms = list(carry[:unroll])
        ss = list(carry[unroll:])
        for u in range(unroll):
            x = buf[pl.ds((i * unroll + u) * L, L)]
            ms[u] = jnp.maximum(ms[u], x)
            ss[u] = ss[u] + x
        return tuple(ms) + tuple(ss)

    mv = functools.reduce(jnp.maximum, pass_a[:unroll])
    S = jnp.sum(functools.reduce(jnp.add, pass_a[unroll:]))
    m = jnp.max(mv)
    mb = jnp.full((L,), m, jnp.float32)

    @plsc.parallel_loop(0, n_it, carry=s_init)
    def pass_b(i, carry):
        es = list(carry)
        for u in range(unroll):
            x = buf[pl.ds((i * unroll + u) * L, L)]
            es[u] = es[u] + jnp.exp(x - mb)
        return tuple(es)

    E = jnp.sum(functools.reduce(jnp.add, pass_b))
    in_shard = label_vec >= 0
    idx = jnp.maximum(label_vec, 0)
    xl_v = jnp.where(in_shard, plsc.load_gather(buf, [idx]), 0.0)
    xl = jnp.max(xl_v)
    return m, S, E, xl


def _store1(res_v, idx, val, lane0):
    plsc.store_scatter(res_v, [jnp.full((L,), idx, jnp.int32)],
                       jnp.full((L,), val, jnp.float32), mask=lane0)


def _sc_body(n_tokens, col0, n_cols, tpw, unroll,
             preds_hbm, labels_hbm, m_hbm, s_hbm, e_hbm, x_hbm,
             row0, row1, lab_v, res_v, sem0, sem1):
    c = lax.axis_index("c")
    s = lax.axis_index("s")
    info = plsc.get_sparse_core_info()
    wid = s * info.num_cores + c
    base = wid * tpw
    lane = jnp.arange(L, dtype=jnp.int32)
    lane0 = lane == 0

    pltpu.sync_copy(labels_hbm.at[pl.ds(base, tpw)], lab_v)

    def row_src(t):
        t = jnp.minimum(t, n_tokens - 1)
        return preds_hbm.at[t, pl.ds(col0, n_cols)]

    pltpu.async_copy(row_src(base), row0, sem0)
    pltpu.async_copy(row_src(base + 1), row1, sem1)

    def handle_token(k, buf, sem, prefetch_t):
        pltpu.make_async_copy(row_src(base + k), buf, sem).wait()
        label_vec = plsc.load_gather(lab_v, [jnp.full((L,), k, jnp.int32)])
        m, S, E, xl = _token_stats(buf, label_vec - col0, n_cols, unroll)
        pltpu.async_copy(row_src(prefetch_t), buf, sem)
        _store1(res_v, k, m, lane0)
        _store1(res_v, tpw + k, S, lane0)
        _store1(res_v, 2 * tpw + k, E, lane0)
        _store1(res_v, 3 * tpw + k, xl, lane0)

    def body(j, carry):
        k0 = 2 * j
        handle_token(k0, row0, sem0, base + k0 + 2)
        handle_token(k0 + 1, row1, sem1, base + k0 + 3)
        return carry

    lax.fori_loop(0, tpw // 2, body, jnp.int32(0))
    pltpu.make_async_copy(row_src(0), row0, sem0).wait()
    pltpu.make_async_copy(row_src(0), row1, sem1).wait()

    for q, out in enumerate((m_hbm, s_hbm, e_hbm, x_hbm)):
        pltpu.sync_copy(res_v.at[pl.ds(q * tpw, tpw)], out.at[pl.ds(base, tpw)])


def _tc_block(preds_ref, labels_ref, m_ref, s_ref, e_ref, x_ref):
    x = preds_ref[...]
    m = jnp.max(x, axis=1)
    e = jnp.sum(jnp.exp(x - m[:, None]), axis=1)
    total = jnp.sum(x, axis=1)
    labels = labels_ref[0, 0, :]
    col = jax.lax.broadcasted_iota(jnp.int32, x.shape, 1)
    xl = jnp.sum(jnp.where(col == labels[:, None], x, 0.0), axis=1)
    m_ref[...] = m[None, None, :]
    s_ref[...] = total[None, None, :]
    e_ref[...] = e[None, None, :]
    x_ref[...] = xl[None, None, :]


def _combine_block(m1_ref, s1_ref, e1_ref, x1_ref, m2_ref, s2_ref, e2_ref,
                   x2_ref, out_ref, *, n_tokens, num_classes):
    m1, m2 = m1_ref[...], m2_ref[...]
    m = jnp.maximum(m1, m2)
    sumexp = e1_ref[...] * jnp.exp(m1 - m) + e2_ref[...] * jnp.exp(m2 - m)
    total = s1_ref[...] + s2_ref[...]
    xl = x1_ref[...] + x2_ref[...]
    lse = m + jnp.log(sumexp)
    a = SMOOTH / (num_classes - 1)
    lp_label = xl - lse
    sum_lp = total - num_classes * lse
    loss_t = -(a * (sum_lp - lp_label) + (1.0 - SMOOTH) * lp_label)
    out_ref[...] = jnp.sum(loss_t, keepdims=True).reshape(1, 1) / n_tokens


def kernel(preds, labels):
    b, t, c = preds.shape
    n_tokens = b * t
    preds2 = preds.reshape(n_tokens, c)
    labels1 = labels.reshape(n_tokens).astype(jnp.int32)

    sc_cols = 10240
    ck = c - sc_cols
    tb = 128

    info = plsc.get_sparse_core_info()
    n_workers = info.num_cores * info.num_subcores
    tpw = n_tokens // n_workers
    unroll = 8

    mesh = plsc.VectorSubcoreMesh(core_axis_name="c", subcore_axis_name="s")
    out_t = jax.ShapeDtypeStruct((n_tokens,), jnp.float32)
    sc = pl.kernel(
        functools.partial(_sc_body, n_tokens, ck, sc_cols, tpw, unroll),
        out_type=(out_t, out_t, out_t, out_t),
        mesh=mesh,
        compiler_params=pltpu.CompilerParams(needs_layout_passes=False),
        scratch_types=[
            pltpu.VMEM((sc_cols,), jnp.float32),
            pltpu.VMEM((sc_cols,), jnp.float32),
            pltpu.VMEM((tpw,), jnp.int32),
            pltpu.VMEM((4 * tpw,), jnp.float32),
            pltpu.SemaphoreType.DMA,
            pltpu.SemaphoreType.DMA,
        ],
    )
    m2_a, s2_a, e2_a, x2_a = sc(preds2, labels1)

    n_blocks = n_tokens // tb
    labels3 = labels1.reshape(n_blocks, 1, tb)
    part_t = jax.ShapeDtypeStruct((n_blocks, 1, tb), jnp.float32)
    part_spec = pl.BlockSpec((1, 1, tb), lambda i: (i, 0, 0))
    m1_a, s1_a, e1_a, x1_a = pl.pallas_call(
        _tc_block,
        grid=(n_blocks,),
        in_specs=[
            pl.BlockSpec((tb, ck), lambda i: (i, 0)),
            pl.BlockSpec((1, 1, tb), lambda i: (i, 0, 0)),
        ],
        out_specs=(part_spec,) * 4,
        out_shape=(part_t,) * 4,
    )(preds2, labels3)

    shp = (n_tokens // 128, 128)
    out = pl.pallas_call(
        functools.partial(_combine_block, n_tokens=n_tokens, num_classes=c),
        out_shape=jax.ShapeDtypeStruct((1, 1), jnp.float32),
    )(m1_a.reshape(shp), s1_a.reshape(shp), e1_a.reshape(shp),
      x1_a.reshape(shp), m2_a.reshape(shp), s2_a.reshape(shp),
      e2_a.reshape(shp), x2_a.reshape(shp))
    return out[0, 0]
